# Initial kernel scaffold; baseline (speedup 1.0000x reference)
import jax
import jax.numpy as jnp
from jax.experimental import pallas as pl


def kernel(x, Wg, W1, b1, W2, b2):
    raise NotImplementedError("write your pallas kernel here")


# speedup vs baseline: 5.1694x; 5.1694x over previous
import functools

import jax
import jax.numpy as jnp
from jax.experimental import pallas as pl
from jax.experimental.pallas import tpu as pltpu

T = 16384
D = 128
H = 256
E = 3


def _moe_body(x_ref, wg_ref, w1_ref, b1_ref, w2_ref, b2_ref, out_ref):
    x = x_ref[...]
    logits = jnp.dot(x, wg_ref[...],
                     preferred_element_type=jnp.float32)
    probs = jax.nn.softmax(logits, axis=-1)
    top_v = jnp.max(probs, axis=-1, keepdims=True)
    top_i = jnp.argmax(probs, axis=-1)
    acc = jnp.zeros_like(x)
    for e in range(E):
        h = jnp.dot(x, w1_ref[e], preferred_element_type=jnp.float32)
        h = jax.nn.gelu(h + b1_ref[e][None, :])
        y = jnp.dot(h, w2_ref[e], preferred_element_type=jnp.float32)
        y = y + b2_ref[e][None, :]
        gate = jnp.where(top_i == e, top_v[:, 0], 0.0)
        acc = acc + gate[:, None] * y
    out_ref[...] = acc


@jax.jit
def kernel(x, Wg, W1, b1, W2, b2):
    B = 1024
    grid = (T // B,)
    return pl.pallas_call(
        _moe_body,
        grid=grid,
        in_specs=[
            pl.BlockSpec((B, D), lambda i: (i, 0)),
            pl.BlockSpec((D, E), lambda i: (0, 0)),
            pl.BlockSpec((E, D, H), lambda i: (0, 0, 0)),
            pl.BlockSpec((E, H), lambda i: (0, 0)),
            pl.BlockSpec((E, H, D), lambda i: (0, 0, 0)),
            pl.BlockSpec((E, D), lambda i: (0, 0)),
        ],
        out_specs=pl.BlockSpec((B, D), lambda i: (i, 0)),
        out_shape=jax.ShapeDtypeStruct((T, D), jnp.float32),
    )(x, Wg, W1, b1, W2, b2)
